# Initial kernel scaffold; baseline (speedup 1.0000x reference)
import jax
import jax.numpy as jnp
from jax.experimental import pallas as pl


def kernel(x, table):
    raise NotImplementedError("write your pallas kernel here")


# speedup vs baseline: 1.4381x; 1.4381x over previous
import functools

import jax
import jax.numpy as jnp
from jax import lax
from jax.experimental import pallas as pl
from jax.experimental.pallas import tpu as pltpu
from jax.experimental.pallas import tpu_sc as plsc

_D = 32
_NC = 2
_NS = 16
_NW = _NC * _NS
_CHUNK = 128


@functools.lru_cache(maxsize=None)
def _build(n_rows: int):
    assert n_rows % (_NW * _CHUNK) == 0
    b_per_w = n_rows // _NW
    n_chunk = b_per_w // _CHUNK
    mesh = plsc.VectorSubcoreMesh(
        core_axis_name="c", subcore_axis_name="s",
        num_cores=_NC, num_subcores=_NS)

    @functools.partial(
        pl.kernel,
        out_type=jax.ShapeDtypeStruct((n_rows, _D), jnp.float32),
        mesh=mesh,
        compiler_params=pltpu.CompilerParams(use_tc_tiling_on_sc=False),
        scratch_types=[
            pltpu.VMEM((b_per_w,), jnp.int32),
            pltpu.VMEM((_CHUNK, _D), jnp.float32),
            pltpu.SemaphoreType.DMA,
        ],
    )
    def gather_kernel(table_hbm, idx_hbm, out_hbm, idx_v, rows_v, sem):
        wid = lax.axis_index("s") * _NC + lax.axis_index("c")
        base = wid * b_per_w
        pltpu.sync_copy(idx_hbm.at[pl.ds(base, b_per_w)], idx_v)

        def chunk_body(c, carry):
            off = c * _CHUNK
            pltpu.async_copy(
                table_hbm.at[idx_v.at[pl.ds(off, _CHUNK)]], rows_v, sem
            ).wait()
            pltpu.sync_copy(rows_v, out_hbm.at[pl.ds(base + off, _CHUNK)])
            return carry

        lax.fori_loop(0, n_chunk, chunk_body, 0)

    return gather_kernel


def kernel(x, table):
    b, f = x.shape
    flat = x.reshape(b * f).astype(jnp.int32)
    out = _build(b * f)(table, flat)
    return out.reshape(b, f, table.shape[1])


# speedup vs baseline: 1.5644x; 1.0878x over previous
import functools

import jax
import jax.numpy as jnp
from jax import lax
from jax.experimental import pallas as pl
from jax.experimental.pallas import tpu as pltpu
from jax.experimental.pallas import tpu_sc as plsc

_D = 32
_NC = 2
_NS = 16
_NW = _NC * _NS
_CHUNK = 128
_NBUF = 4
_NOBUF = 2
_BIG = _CHUNK * _D // 128


@functools.lru_cache(maxsize=None)
def _build(n_rows: int):
    assert n_rows % (_NW * _CHUNK) == 0
    b_per_w = n_rows // _NW
    n_chunk = b_per_w // _CHUNK
    n_group = n_chunk // _NBUF
    mesh = plsc.VectorSubcoreMesh(
        core_axis_name="c", subcore_axis_name="s",
        num_cores=_NC, num_subcores=_NS)

    @functools.partial(
        pl.kernel,
        out_type=jax.ShapeDtypeStruct((n_rows * _D // 128, 128), jnp.float32),
        mesh=mesh,
        compiler_params=pltpu.CompilerParams(use_tc_tiling_on_sc=False),
        scratch_types=[
            pltpu.VMEM((b_per_w,), jnp.int32),
            pltpu.VMEM((_NBUF, _CHUNK, _D), jnp.float32),
            pltpu.VMEM((_NOBUF, _BIG, 128), jnp.float32),
        ] + [pltpu.SemaphoreType.DMA] * (_NBUF + _NOBUF),
    )
    def gather_kernel(table_hbm, idx_hbm, out_hbm, idx_v, gbuf, obuf,
                      g0, g1, g2, g3, s0, s1):
        gsems = (g0, g1, g2, g3)
        ssems = (s0, s1)
        wid = lax.axis_index("s") * _NC + lax.axis_index("c")
        base = wid * b_per_w
        big_base = base * _D // 128
        pltpu.sync_copy(idx_hbm.at[pl.ds(base, b_per_w)], idx_v)

        def fire_gather(c, u):
            pltpu.async_copy(
                table_hbm.at[idx_v.at[pl.ds(c * _CHUNK, _CHUNK)]],
                gbuf.at[u], gsems[u])

        for u in range(_NBUF - 1):
            fire_gather(u, u)

        def group_body(g, carry):
            for u in range(_NBUF):
                o = u % _NOBUF
                c = g * _NBUF + u

                pltpu.make_async_copy(
                    table_hbm.at[pl.ds(0, _CHUNK)], gbuf.at[u],
                    gsems[u]).wait()

                @pl.when(c + _NBUF - 1 < n_chunk)
                def _():
                    fire_gather(c + _NBUF - 1, (u - 1) % _NBUF)

                @pl.when(c >= _NOBUF)
                def _():
                    pltpu.make_async_copy(
                        obuf.at[o], out_hbm.at[pl.ds(0, _BIG)],
                        ssems[o]).wait()

                src = gbuf.at[u]
                dst = obuf.at[o]
                for j in range(_CHUNK):
                    for k in range(_D // 16):
                        p = j * _D + k * 16
                        dst[p // 128, pl.ds(p % 128, 16)] = (
                            src[j, pl.ds(k * 16, 16)])

                pltpu.async_copy(
                    obuf.at[o],
                    out_hbm.at[pl.ds(big_base + c * _BIG, _BIG)],
                    ssems[o])
            return carry

        lax.fori_loop(0, n_group, group_body, 0)

        for o in range(_NOBUF):
            pltpu.make_async_copy(
                obuf.at[o], out_hbm.at[pl.ds(0, _BIG)], ssems[o]).wait()

    return gather_kernel


def kernel(x, table):
    b, f = x.shape
    n_vocab, d = table.shape
    flat = x.reshape(b * f).astype(jnp.int32)
    out = _build(b * f)(table, flat)
    return out.reshape(b, f, d)
